# Initial kernel scaffold; baseline (speedup 1.0000x reference)
import jax
import jax.numpy as jnp
from jax.experimental import pallas as pl


def kernel(x, edge_index, feature_importance, W1, b1, gamma, beta, running_mean, running_var, W2, b2):
    raise NotImplementedError("write your pallas kernel here")


# speedup vs baseline: 25.7909x; 25.7909x over previous
import functools

import jax
import jax.numpy as jnp
from jax import lax
from jax.experimental import pallas as pl
from jax.experimental.pallas import tpu as pltpu
from jax.experimental.pallas import tpu_sc as plsc

NC = 2
NS = 16
NW = NC * NS
LANES = 16


def _sc_mesh():
    return plsc.VectorSubcoreMesh(core_axis_name="c", subcore_axis_name="s")


def _make_deg_kernel(npad, e, k=512):
    K = k
    nchunks = e // K
    assert nchunks * K == e
    base_chunks, extra = divmod(nchunks, NW)

    @functools.partial(
        pl.kernel,
        mesh=_sc_mesh(),
        out_type=jax.ShapeDtypeStruct((NW * npad,), jnp.float32),
        scratch_types=[
            pltpu.VMEM((K,), jnp.int32),
            pltpu.VMEM((npad,), jnp.float32),
        ],
        compiler_params=pltpu.CompilerParams(needs_layout_passes=False),
    )
    def deg_kernel(dst_hbm, out_hbm, dst_v, deg_v):
        cid = lax.axis_index("c")
        sid = lax.axis_index("s")
        wid = sid * NC + cid
        zeros16 = jnp.zeros((LANES,), jnp.float32)
        ones16 = jnp.ones((LANES,), jnp.float32)

        def zero_body(i, carry):
            deg_v[pl.ds(i * LANES, LANES)] = zeros16
            return carry

        lax.fori_loop(0, npad // LANES, zero_body, 0)

        def chunk_body(i, carry):
            off = (i * NW + wid) * K
            pltpu.sync_copy(dst_hbm.at[pl.ds(off, K)], dst_v)

            def acc_body(j, c2):
                idx = dst_v[pl.ds(j * LANES, LANES)]
                plsc.addupdate_scatter(deg_v, [idx], ones16)
                return c2

            lax.fori_loop(0, K // LANES, acc_body, 0)
            return carry

        nch = base_chunks + jnp.where(wid < extra, 1, 0)
        lax.fori_loop(0, nch, chunk_body, 0)
        pltpu.sync_copy(deg_v, out_hbm.at[pl.ds(wid * npad, npad)])

    return deg_kernel


def _make_agg_kernel(npad, e, w, k):
    K = k
    nchunks = e // K
    assert nchunks * K == e
    base_chunks, extra = divmod(nchunks, NW)
    rows_per_tile = npad // NS
    assert rows_per_tile * NS == npad and rows_per_tile % 8 == 0

    @functools.partial(
        pl.kernel,
        mesh=_sc_mesh(),
        out_type=jax.ShapeDtypeStruct((NC, npad, w), jnp.float32),
        scratch_types=[
            pltpu.VMEM((K,), jnp.int32),
            pltpu.VMEM((K,), jnp.int32),
            pltpu.VMEM((K, w), jnp.float32),
            pltpu.VMEM_SHARED((npad, w), jnp.float32),
            pltpu.SemaphoreType.DMA,
        ],
        compiler_params=pltpu.CompilerParams(
            needs_layout_passes=False,
            use_tc_tiling_on_sc=(w % 128 == 0),
        ),
    )
    def agg_kernel(table_hbm, src_hbm, dst_hbm, out_hbm,
                   src_v, dst_v, rows_v, accum_sh, sem):
        cid = lax.axis_index("c")
        sid = lax.axis_index("s")
        wid = sid * NC + cid
        stripe = pl.ds(sid * rows_per_tile, rows_per_tile)
        pltpu.sync_copy(table_hbm.at[stripe], accum_sh.at[stripe])
        plsc.subcore_barrier()

        def chunk_body(i, carry):
            off = (i * NW + wid) * K
            pltpu.sync_copy(src_hbm.at[pl.ds(off, K)], src_v)
            pltpu.sync_copy(dst_hbm.at[pl.ds(off, K)], dst_v)
            pltpu.async_copy(table_hbm.at[src_v], rows_v, sem).wait()
            pltpu.sync_copy(rows_v, accum_sh.at[dst_v], add=True)
            return carry

        nch = base_chunks + jnp.where(wid < extra, 1, 0)
        lax.fori_loop(0, nch, chunk_body, 0)
        plsc.subcore_barrier()
        pltpu.sync_copy(accum_sh.at[stripe], out_hbm.at[cid, stripe])

    return agg_kernel


def _dis_body(degp_ref, out_ref):
    deg = jnp.sum(degp_ref[...], axis=0) + 1.0
    out_ref[...] = lax.rsqrt(deg)[:, None]


def _prep_body(x_ref, fi_ref, w1_ref, dis_ref, out_ref):
    xw = x_ref[...] * jax.nn.sigmoid(fi_ref[...])[0][None, :]
    h = jnp.dot(xw, w1_ref[...], preferred_element_type=jnp.float32)
    out_ref[...] = h * dis_ref[...]


def _mid_body(agg_ref, table_ref, dis_ref, b1_ref,
              bnw_ref, bnb_ref, w2_ref, out_ref):
    dis = dis_ref[...]
    agg = agg_ref[0] + agg_ref[1] - table_ref[...]
    t = dis * agg + b1_ref[...][0][None, :]
    t = t * bnw_ref[...][0][None, :] + bnb_ref[...][0][None, :]
    t = jnp.maximum(t, 0.0)
    g = jnp.dot(t, w2_ref[...], preferred_element_type=jnp.float32)
    out_ref[...] = g * dis


def _final_body(agg_ref, table_ref, dis_ref, b2_ref, out_ref):
    agg = agg_ref[0] + agg_ref[1] - table_ref[...]
    out_ref[...] = dis_ref[...] * agg + b2_ref[...][0][None, :]


W2PAD = 16


def kernel(x, edge_index, feature_importance, W1, b1, gamma, beta,
           running_mean, running_var, W2, b2):
    n, d = x.shape
    e = edge_index.shape[1]
    h = W1.shape[1]
    out_dim = W2.shape[1]
    src = edge_index[0].astype(jnp.int32)
    dst = edge_index[1].astype(jnp.int32)

    row_align = NS * 8 * 8
    npad = ((n + row_align - 1) // row_align) * row_align
    xp = jnp.zeros((npad, d), x.dtype).at[:n].set(x)

    degp = _make_deg_kernel(npad, e)(dst).reshape(NW, npad)

    bn = 2048
    grid = npad // bn

    dis2d = pl.pallas_call(
        _dis_body,
        out_shape=jax.ShapeDtypeStruct((npad, 1), jnp.float32),
    )(degp)

    h1p = pl.pallas_call(
        _prep_body,
        grid=(grid,),
        in_specs=[
            pl.BlockSpec((bn, d), lambda i: (i, 0)),
            pl.BlockSpec((1, d), lambda i: (0, 0)),
            pl.BlockSpec((d, h), lambda i: (0, 0)),
            pl.BlockSpec((bn, 1), lambda i: (i, 0)),
        ],
        out_specs=pl.BlockSpec((bn, h), lambda i: (i, 0)),
        out_shape=jax.ShapeDtypeStruct((npad, h), jnp.float32),
    )(xp, feature_importance[None, :], W1, dis2d)

    agg1 = _make_agg_kernel(npad, e, h, 256)(h1p, src, dst)

    w2p = jnp.zeros((h, W2PAD), jnp.float32).at[:, :out_dim].set(W2)
    bnw = gamma * lax.rsqrt(running_var + 1e-5)
    bnb = beta - running_mean * bnw
    gp = pl.pallas_call(
        _mid_body,
        grid=(grid,),
        in_specs=[
            pl.BlockSpec((NC, bn, h), lambda i: (0, i, 0)),
            pl.BlockSpec((bn, h), lambda i: (i, 0)),
            pl.BlockSpec((bn, 1), lambda i: (i, 0)),
            pl.BlockSpec((1, h), lambda i: (0, 0)),
            pl.BlockSpec((1, h), lambda i: (0, 0)),
            pl.BlockSpec((1, h), lambda i: (0, 0)),
            pl.BlockSpec((h, W2PAD), lambda i: (0, 0)),
        ],
        out_specs=pl.BlockSpec((bn, W2PAD), lambda i: (i, 0)),
        out_shape=jax.ShapeDtypeStruct((npad, W2PAD), jnp.float32),
    )(agg1, h1p, dis2d, b1[None, :], bnw[None, :], bnb[None, :], w2p)

    agg2 = _make_agg_kernel(npad, e, W2PAD, 512)(gp, src, dst)

    b2p = jnp.zeros((W2PAD,), jnp.float32).at[:out_dim].set(b2)
    outp = pl.pallas_call(
        _final_body,
        grid=(grid,),
        in_specs=[
            pl.BlockSpec((NC, bn, W2PAD), lambda i: (0, i, 0)),
            pl.BlockSpec((bn, W2PAD), lambda i: (i, 0)),
            pl.BlockSpec((bn, 1), lambda i: (i, 0)),
            pl.BlockSpec((1, W2PAD), lambda i: (0, 0)),
        ],
        out_specs=pl.BlockSpec((bn, W2PAD), lambda i: (i, 0)),
        out_shape=jax.ShapeDtypeStruct((npad, W2PAD), jnp.float32),
    )(agg2, gp, dis2d, b2p[None, :])

    return outp[:n, :out_dim]


# speedup vs baseline: 36.4560x; 1.4135x over previous
import functools

import jax
import jax.numpy as jnp
from jax import lax
from jax.experimental import pallas as pl
from jax.experimental.pallas import tpu as pltpu
from jax.experimental.pallas import tpu_sc as plsc

NC = 2
NS = 16
NW = NC * NS
LANES = 16


def _sc_mesh():
    return plsc.VectorSubcoreMesh(core_axis_name="c", subcore_axis_name="s")


def _make_deg_kernel(npad, e):
    ew = e // NW
    assert ew * NW == e and ew % LANES == 0

    @functools.partial(
        pl.kernel,
        mesh=_sc_mesh(),
        out_type=jax.ShapeDtypeStruct((NW * npad,), jnp.float32),
        scratch_types=[
            pltpu.VMEM((ew,), jnp.int32),
            pltpu.VMEM((npad,), jnp.float32),
        ],
        compiler_params=pltpu.CompilerParams(needs_layout_passes=False),
    )
    def deg_kernel(dst_hbm, out_hbm, dst_v, deg_v):
        cid = lax.axis_index("c")
        sid = lax.axis_index("s")
        wid = sid * NC + cid
        zeros16 = jnp.zeros((LANES,), jnp.float32)
        ones16 = jnp.ones((LANES,), jnp.float32)

        def zero_body(i, carry):
            deg_v[pl.ds(i * LANES, LANES)] = zeros16
            return carry

        lax.fori_loop(0, npad // LANES, zero_body, 0)
        pltpu.sync_copy(dst_hbm.at[pl.ds(wid * ew, ew)], dst_v)

        def acc_body(j, c2):
            idx = dst_v[pl.ds(j * LANES, LANES)]
            plsc.addupdate_scatter(deg_v, [idx], ones16)
            return c2

        lax.fori_loop(0, ew // LANES, acc_body, 0)
        pltpu.sync_copy(deg_v, out_hbm.at[pl.ds(wid * npad, npad)])

    return deg_kernel


def _make_agg_kernel(npad, e, w, k, sb):
    cs = e // (NW * k * sb)
    assert cs * NW * k * sb == e
    assert cs % 2 == 1 and cs >= 3
    pairs = (cs - 1) // 2
    rows_per_tile = npad // NS
    assert rows_per_tile * NS == npad and rows_per_tile % 8 == 0

    @functools.partial(
        pl.kernel,
        mesh=_sc_mesh(),
        out_type=jax.ShapeDtypeStruct((NC, npad, w), jnp.float32),
        scratch_types=[
            pltpu.VMEM((cs, k), jnp.int32),
            pltpu.VMEM((cs, k), jnp.int32),
            pltpu.VMEM((k, w), jnp.float32),
            pltpu.VMEM((k, w), jnp.float32),
            pltpu.VMEM_SHARED((npad, w), jnp.float32),
            pltpu.SemaphoreType.DMA,
            pltpu.SemaphoreType.DMA,
        ],
        compiler_params=pltpu.CompilerParams(
            needs_layout_passes=False,
            use_tc_tiling_on_sc=(w % 128 == 0),
        ),
    )
    def agg_kernel(table_hbm, src_hbm, dst_hbm, out_hbm,
                   src_v, dst_v, rows_a, rows_b, accum_sh, ga, gb):
        cid = lax.axis_index("c")
        sid = lax.axis_index("s")
        wid = sid * NC + cid
        stripe = pl.ds(sid * rows_per_tile, rows_per_tile)
        pltpu.sync_copy(table_hbm.at[stripe], accum_sh.at[stripe])
        plsc.subcore_barrier()

        def gather(c, rows, sem):
            pltpu.async_copy(table_hbm.at[src_v.at[c]], rows, sem)

        def gwait(c, rows, sem):
            pltpu.make_async_copy(table_hbm.at[src_v.at[c]], rows, sem).wait()

        def scat(c, rows):
            pltpu.sync_copy(rows, accum_sh.at[dst_v.at[c]], add=True)

        def subblock(s, carry):
            pltpu.sync_copy(src_hbm.at[wid, s], src_v)
            pltpu.sync_copy(dst_hbm.at[wid, s], dst_v)
            gather(0, rows_a, ga)

            def pair_body(j, c2):
                c0 = 2 * j
                gather(c0 + 1, rows_b, gb)
                gwait(c0, rows_a, ga)
                scat(c0, rows_a)
                gather(c0 + 2, rows_a, ga)
                gwait(c0 + 1, rows_b, gb)
                scat(c0 + 1, rows_b)
                return c2

            lax.fori_loop(0, pairs, pair_body, 0)
            gwait(cs - 1, rows_a, ga)
            scat(cs - 1, rows_a)
            return carry

        lax.fori_loop(0, sb, subblock, 0)
        plsc.subcore_barrier()
        pltpu.sync_copy(accum_sh.at[stripe], out_hbm.at[cid, stripe])

    return agg_kernel


def _dis_body(degp_ref, out_ref):
    deg = jnp.sum(degp_ref[...], axis=0) + 1.0
    out_ref[...] = lax.rsqrt(deg)[:, None]


def _prep_body(x_ref, fi_ref, w1_ref, dis_ref, out_ref):
    xw = x_ref[...] * jax.nn.sigmoid(fi_ref[...])[0][None, :]
    h = jnp.dot(xw, w1_ref[...], preferred_element_type=jnp.float32)
    out_ref[...] = h * dis_ref[...]


def _mid_body(agg_ref, table_ref, dis_ref, b1_ref,
              bnw_ref, bnb_ref, w2_ref, out_ref):
    dis = dis_ref[...]
    agg = agg_ref[0] + agg_ref[1] - table_ref[...]
    t = dis * agg + b1_ref[...][0][None, :]
    t = t * bnw_ref[...][0][None, :] + bnb_ref[...][0][None, :]
    t = jnp.maximum(t, 0.0)
    g = jnp.dot(t, w2_ref[...], preferred_element_type=jnp.float32)
    out_ref[...] = g * dis


def _final_body(agg_ref, table_ref, dis_ref, b2_ref, out_ref):
    agg = agg_ref[0] + agg_ref[1] - table_ref[...]
    out_ref[...] = dis_ref[...] * agg + b2_ref[...][0][None, :]


W2PAD = 16


def kernel(x, edge_index, feature_importance, W1, b1, gamma, beta,
           running_mean, running_var, W2, b2):
    n, d = x.shape
    e = edge_index.shape[1]
    h = W1.shape[1]
    out_dim = W2.shape[1]
    src = edge_index[0].astype(jnp.int32)
    dst = edge_index[1].astype(jnp.int32)

    row_align = NS * 8 * 8
    npad = ((n + row_align - 1) // row_align) * row_align
    xp = jnp.zeros((npad, d), x.dtype).at[:n].set(x)

    k1, sb1 = 80, 5
    k2, sb2 = 2000, 1
    src1 = src.reshape(NW, sb1, e // (NW * k1 * sb1), k1)
    dst1 = dst.reshape(NW, sb1, e // (NW * k1 * sb1), k1)
    src2 = src.reshape(NW, sb2, e // (NW * k2 * sb2), k2)
    dst2 = dst.reshape(NW, sb2, e // (NW * k2 * sb2), k2)

    degp = _make_deg_kernel(npad, e)(dst).reshape(NW, npad)

    bn = 2048
    grid = npad // bn

    dis2d = pl.pallas_call(
        _dis_body,
        out_shape=jax.ShapeDtypeStruct((npad, 1), jnp.float32),
    )(degp)

    h1p = pl.pallas_call(
        _prep_body,
        grid=(grid,),
        in_specs=[
            pl.BlockSpec((bn, d), lambda i: (i, 0)),
            pl.BlockSpec((1, d), lambda i: (0, 0)),
            pl.BlockSpec((d, h), lambda i: (0, 0)),
            pl.BlockSpec((bn, 1), lambda i: (i, 0)),
        ],
        out_specs=pl.BlockSpec((bn, h), lambda i: (i, 0)),
        out_shape=jax.ShapeDtypeStruct((npad, h), jnp.float32),
    )(xp, feature_importance[None, :], W1, dis2d)

    agg1 = _make_agg_kernel(npad, e, h, k1, sb1)(h1p, src1, dst1)

    w2p = jnp.zeros((h, W2PAD), jnp.float32).at[:, :out_dim].set(W2)
    bnw = gamma * lax.rsqrt(running_var + 1e-5)
    bnb = beta - running_mean * bnw
    gp = pl.pallas_call(
        _mid_body,
        grid=(grid,),
        in_specs=[
            pl.BlockSpec((NC, bn, h), lambda i: (0, i, 0)),
            pl.BlockSpec((bn, h), lambda i: (i, 0)),
            pl.BlockSpec((bn, 1), lambda i: (i, 0)),
            pl.BlockSpec((1, h), lambda i: (0, 0)),
            pl.BlockSpec((1, h), lambda i: (0, 0)),
            pl.BlockSpec((1, h), lambda i: (0, 0)),
            pl.BlockSpec((h, W2PAD), lambda i: (0, 0)),
        ],
        out_specs=pl.BlockSpec((bn, W2PAD), lambda i: (i, 0)),
        out_shape=jax.ShapeDtypeStruct((npad, W2PAD), jnp.float32),
    )(agg1, h1p, dis2d, b1[None, :], bnw[None, :], bnb[None, :], w2p)

    agg2 = _make_agg_kernel(npad, e, W2PAD, k2, sb2)(gp, src2, dst2)

    b2p = jnp.zeros((W2PAD,), jnp.float32).at[:out_dim].set(b2)
    outp = pl.pallas_call(
        _final_body,
        grid=(grid,),
        in_specs=[
            pl.BlockSpec((NC, bn, W2PAD), lambda i: (0, i, 0)),
            pl.BlockSpec((bn, W2PAD), lambda i: (i, 0)),
            pl.BlockSpec((bn, 1), lambda i: (i, 0)),
            pl.BlockSpec((1, W2PAD), lambda i: (0, 0)),
        ],
        out_specs=pl.BlockSpec((bn, W2PAD), lambda i: (i, 0)),
        out_shape=jax.ShapeDtypeStruct((npad, W2PAD), jnp.float32),
    )(agg2, gp, dis2d, b2p[None, :])

    return outp[:n, :out_dim]


# speedup vs baseline: 37.4048x; 1.0260x over previous
import functools

import jax
import jax.numpy as jnp
from jax import lax
from jax.experimental import pallas as pl
from jax.experimental.pallas import tpu as pltpu
from jax.experimental.pallas import tpu_sc as plsc

NC = 2
NS = 16
NW = NC * NS
LANES = 16


def _sc_mesh():
    return plsc.VectorSubcoreMesh(core_axis_name="c", subcore_axis_name="s")


def _make_deg_kernel(npad, e):
    ew = e // NW
    assert ew * NW == e and ew % LANES == 0

    @functools.partial(
        pl.kernel,
        mesh=_sc_mesh(),
        out_type=jax.ShapeDtypeStruct((NW * npad,), jnp.float32),
        scratch_types=[
            pltpu.VMEM((ew,), jnp.int32),
            pltpu.VMEM((npad,), jnp.float32),
        ],
        compiler_params=pltpu.CompilerParams(needs_layout_passes=False),
    )
    def deg_kernel(dst_hbm, out_hbm, dst_v, deg_v):
        cid = lax.axis_index("c")
        sid = lax.axis_index("s")
        wid = sid * NC + cid
        zeros16 = jnp.zeros((LANES,), jnp.float32)
        ones16 = jnp.ones((LANES,), jnp.float32)

        def zero_body(i, carry):
            deg_v[pl.ds(i * LANES, LANES)] = zeros16
            return carry

        lax.fori_loop(0, npad // LANES, zero_body, 0)
        pltpu.sync_copy(dst_hbm.at[pl.ds(wid * ew, ew)], dst_v)

        def acc_body(j, c2):
            idx = dst_v[pl.ds(j * LANES, LANES)]
            plsc.addupdate_scatter(deg_v, [idx], ones16)
            return c2

        lax.fori_loop(0, ew // LANES, acc_body, 0)
        pltpu.sync_copy(deg_v, out_hbm.at[pl.ds(wid * npad, npad)])

    return deg_kernel


def _make_agg_kernel(npad, e, w, k, sb):
    cs = e // (NW * k * sb)
    assert cs * NW * k * sb == e
    assert cs % 2 == 1 and cs >= 3
    pairs = (cs - 1) // 2
    rows_per_tile = npad // NS
    assert rows_per_tile * NS == npad and rows_per_tile % 8 == 0

    @functools.partial(
        pl.kernel,
        mesh=_sc_mesh(),
        out_type=jax.ShapeDtypeStruct((NC, npad, w), jnp.float32),
        scratch_types=[
            pltpu.VMEM((cs, k), jnp.int32),
            pltpu.VMEM((cs, k), jnp.int32),
            pltpu.VMEM((k, w), jnp.float32),
            pltpu.VMEM((k, w), jnp.float32),
            pltpu.VMEM_SHARED((npad, w), jnp.float32),
            pltpu.SemaphoreType.DMA,
            pltpu.SemaphoreType.DMA,
        ],
        compiler_params=pltpu.CompilerParams(
            needs_layout_passes=False,
            use_tc_tiling_on_sc=(w % 128 == 0),
        ),
    )
    def agg_kernel(table_hbm, src_hbm, dst_hbm, out_hbm,
                   src_v, dst_v, rows_a, rows_b, accum_sh, ga, gb):
        cid = lax.axis_index("c")
        sid = lax.axis_index("s")
        wid = sid * NC + cid
        stripe = pl.ds(sid * rows_per_tile, rows_per_tile)
        pltpu.sync_copy(table_hbm.at[stripe], accum_sh.at[stripe])
        plsc.subcore_barrier()

        def gather(c, rows, sem):
            pltpu.async_copy(table_hbm.at[src_v.at[c]], rows, sem)

        def gwait(c, rows, sem):
            pltpu.make_async_copy(table_hbm.at[src_v.at[c]], rows, sem).wait()

        def scat(c, rows):
            pltpu.sync_copy(rows, accum_sh.at[dst_v.at[c]], add=True)

        def subblock(s, carry):
            pltpu.sync_copy(src_hbm.at[wid, s], src_v)
            pltpu.sync_copy(dst_hbm.at[wid, s], dst_v)
            gather(0, rows_a, ga)

            def pair_body(j, c2):
                c0 = 2 * j
                gather(c0 + 1, rows_b, gb)
                gwait(c0, rows_a, ga)
                scat(c0, rows_a)
                gather(c0 + 2, rows_a, ga)
                gwait(c0 + 1, rows_b, gb)
                scat(c0 + 1, rows_b)
                return c2

            lax.fori_loop(0, pairs, pair_body, 0)
            gwait(cs - 1, rows_a, ga)
            scat(cs - 1, rows_a)
            return carry

        lax.fori_loop(0, sb, subblock, 0)
        plsc.subcore_barrier()
        pltpu.sync_copy(accum_sh.at[stripe], out_hbm.at[cid, stripe])

    return agg_kernel


def _dis(degp_blk):
    deg = jnp.sum(degp_blk, axis=0) + 1.0
    return lax.rsqrt(deg)[:, None]


def _prep_body(x_ref, fi_ref, w1_ref, degp_ref, out_ref):
    xw = x_ref[...] * jax.nn.sigmoid(fi_ref[...])[0][None, :]
    h = jnp.dot(xw, w1_ref[...], preferred_element_type=jnp.float32)
    out_ref[...] = h * _dis(degp_ref[...])


def _mid_body(agg_ref, table_ref, degp_ref, b1_ref,
              bnw_ref, bnb_ref, w2_ref, out_ref):
    dis = _dis(degp_ref[...])
    agg = agg_ref[0] + agg_ref[1] - table_ref[...]
    t = dis * agg + b1_ref[...][0][None, :]
    t = t * bnw_ref[...][0][None, :] + bnb_ref[...][0][None, :]
    t = jnp.maximum(t, 0.0)
    g = jnp.dot(t, w2_ref[...], preferred_element_type=jnp.float32)
    out_ref[...] = g * dis


def _final_body(agg_ref, table_ref, degp_ref, b2_ref, out_ref):
    agg = agg_ref[0] + agg_ref[1] - table_ref[...]
    out_ref[...] = _dis(degp_ref[...]) * agg + b2_ref[...][0][None, :]


W2PAD = 16


def kernel(x, edge_index, feature_importance, W1, b1, gamma, beta,
           running_mean, running_var, W2, b2):
    n, d = x.shape
    e = edge_index.shape[1]
    h = W1.shape[1]
    out_dim = W2.shape[1]
    src = edge_index[0].astype(jnp.int32)
    dst = edge_index[1].astype(jnp.int32)

    row_align = NS * 8 * 8
    npad = ((n + row_align - 1) // row_align) * row_align

    k1, sb1 = 80, 5
    k2, sb2 = 2000, 1
    src1 = src.reshape(NW, sb1, e // (NW * k1 * sb1), k1)
    dst1 = dst.reshape(NW, sb1, e // (NW * k1 * sb1), k1)
    src2 = src.reshape(NW, sb2, e // (NW * k2 * sb2), k2)
    dst2 = dst.reshape(NW, sb2, e // (NW * k2 * sb2), k2)

    degp = _make_deg_kernel(npad, e)(dst).reshape(NW, npad)

    bn = 2048
    grid = npad // bn

    h1p = pl.pallas_call(
        _prep_body,
        grid=(grid,),
        in_specs=[
            pl.BlockSpec((bn, d), lambda i: (i, 0)),
            pl.BlockSpec((1, d), lambda i: (0, 0)),
            pl.BlockSpec((d, h), lambda i: (0, 0)),
            pl.BlockSpec((NW, bn), lambda i: (0, i)),
        ],
        out_specs=pl.BlockSpec((bn, h), lambda i: (i, 0)),
        out_shape=jax.ShapeDtypeStruct((npad, h), jnp.float32),
    )(x, feature_importance[None, :], W1, degp)

    agg1 = _make_agg_kernel(npad, e, h, k1, sb1)(h1p, src1, dst1)

    w2p = jnp.zeros((h, W2PAD), jnp.float32).at[:, :out_dim].set(W2)
    bnw = gamma * lax.rsqrt(running_var + 1e-5)
    bnb = beta - running_mean * bnw
    gp = pl.pallas_call(
        _mid_body,
        grid=(grid,),
        in_specs=[
            pl.BlockSpec((NC, bn, h), lambda i: (0, i, 0)),
            pl.BlockSpec((bn, h), lambda i: (i, 0)),
            pl.BlockSpec((NW, bn), lambda i: (0, i)),
            pl.BlockSpec((1, h), lambda i: (0, 0)),
            pl.BlockSpec((1, h), lambda i: (0, 0)),
            pl.BlockSpec((1, h), lambda i: (0, 0)),
            pl.BlockSpec((h, W2PAD), lambda i: (0, 0)),
        ],
        out_specs=pl.BlockSpec((bn, W2PAD), lambda i: (i, 0)),
        out_shape=jax.ShapeDtypeStruct((npad, W2PAD), jnp.float32),
    )(agg1, h1p, degp, b1[None, :], bnw[None, :], bnb[None, :], w2p)

    agg2 = _make_agg_kernel(npad, e, W2PAD, k2, sb2)(gp, src2, dst2)

    b2p = jnp.zeros((W2PAD,), jnp.float32).at[:out_dim].set(b2)
    outp = pl.pallas_call(
        _final_body,
        grid=(grid,),
        in_specs=[
            pl.BlockSpec((NC, bn, W2PAD), lambda i: (0, i, 0)),
            pl.BlockSpec((bn, W2PAD), lambda i: (i, 0)),
            pl.BlockSpec((NW, bn), lambda i: (0, i)),
            pl.BlockSpec((1, W2PAD), lambda i: (0, 0)),
        ],
        out_specs=pl.BlockSpec((bn, W2PAD), lambda i: (i, 0)),
        out_shape=jax.ShapeDtypeStruct((npad, W2PAD), jnp.float32),
    )(agg2, gp, degp, b2p[None, :])

    return outp[:n, :out_dim]


# speedup vs baseline: 38.4812x; 1.0288x over previous
import functools

import jax
import jax.numpy as jnp
from jax import lax
from jax.experimental import pallas as pl
from jax.experimental.pallas import tpu as pltpu
from jax.experimental.pallas import tpu_sc as plsc

NC = 2
NS = 16
NW = NC * NS
LANES = 16


def _sc_mesh():
    return plsc.VectorSubcoreMesh(core_axis_name="c", subcore_axis_name="s")


def _make_deg_kernel(npad, e):
    ew = e // NW
    assert ew * NW == e and ew % LANES == 0

    @functools.partial(
        pl.kernel,
        mesh=_sc_mesh(),
        out_type=jax.ShapeDtypeStruct((NW * npad,), jnp.float32),
        scratch_types=[
            pltpu.VMEM((ew,), jnp.int32),
            pltpu.VMEM((npad,), jnp.float32),
        ],
        compiler_params=pltpu.CompilerParams(needs_layout_passes=False),
    )
    def deg_kernel(dst_hbm, out_hbm, dst_v, deg_v):
        cid = lax.axis_index("c")
        sid = lax.axis_index("s")
        wid = sid * NC + cid
        zeros16 = jnp.zeros((LANES,), jnp.float32)
        ones16 = jnp.ones((LANES,), jnp.float32)

        def zero_body(i, carry):
            deg_v[pl.ds(i * LANES, LANES)] = zeros16
            return carry

        lax.fori_loop(0, npad // LANES, zero_body, 0)
        pltpu.sync_copy(dst_hbm.at[pl.ds(wid * ew, ew)], dst_v)

        def acc_body(j, c2):
            idx = dst_v[pl.ds(j * LANES, LANES)]
            plsc.addupdate_scatter(deg_v, [idx], ones16)
            return c2

        lax.fori_loop(0, ew // LANES, acc_body, 0)
        pltpu.sync_copy(deg_v, out_hbm.at[pl.ds(wid * npad, npad)])

    return deg_kernel


def _make_agg_kernel(npad, e, w, k, sb):
    cs = e // (NW * k * sb)
    assert cs * NW * k * sb == e
    assert cs % 2 == 1 and cs >= 3
    pairs = (cs - 1) // 2
    rows_per_tile = npad // NS
    assert rows_per_tile * NS == npad and rows_per_tile % 8 == 0

    @functools.partial(
        pl.kernel,
        mesh=_sc_mesh(),
        out_type=jax.ShapeDtypeStruct((NC, npad, w), jnp.float32),
        scratch_types=[
            pltpu.VMEM((2, cs, k), jnp.int32),
            pltpu.VMEM((2, cs, k), jnp.int32),
            pltpu.VMEM((k, w), jnp.float32),
            pltpu.VMEM((k, w), jnp.float32),
            pltpu.VMEM_SHARED((npad, w), jnp.float32),
            pltpu.SemaphoreType.DMA,
            pltpu.SemaphoreType.DMA,
            pltpu.SemaphoreType.DMA,
        ],
        compiler_params=pltpu.CompilerParams(
            needs_layout_passes=False,
            use_tc_tiling_on_sc=(w % 128 == 0),
        ),
    )
    def agg_kernel(table_hbm, src_hbm, dst_hbm, out_hbm,
                   src_v, dst_v, rows_a, rows_b, accum_sh, ga, gb, isem):
        cid = lax.axis_index("c")
        sid = lax.axis_index("s")
        wid = sid * NC + cid
        stripe = pl.ds(sid * rows_per_tile, rows_per_tile)

        def idx_start(s, buf):
            pltpu.async_copy(src_hbm.at[wid, s], src_v.at[buf], isem)
            pltpu.async_copy(dst_hbm.at[wid, s], dst_v.at[buf], isem)

        def idx_wait(s, buf):
            pltpu.make_async_copy(src_hbm.at[wid, s], src_v.at[buf], isem).wait()
            pltpu.make_async_copy(dst_hbm.at[wid, s], dst_v.at[buf], isem).wait()

        def gather(buf, c, rows, sem):
            pltpu.async_copy(table_hbm.at[src_v.at[buf, c]], rows, sem)

        def gwait(buf, c, rows, sem):
            pltpu.make_async_copy(
                table_hbm.at[src_v.at[buf, c]], rows, sem).wait()

        def scat(buf, c, rows):
            pltpu.sync_copy(rows, accum_sh.at[dst_v.at[buf, c]], add=True)

        idx_start(0, 0)
        pltpu.sync_copy(table_hbm.at[stripe], accum_sh.at[stripe])
        plsc.subcore_barrier()
        idx_wait(0, 0)
        gather(0, 0, rows_a, ga)

        for s in range(sb):
            cur = s % 2
            if s + 1 < sb:
                idx_start(s + 1, 1 - cur)

            def pair_body(j, c2, cur=cur):
                c0 = 2 * j
                gather(cur, c0 + 1, rows_b, gb)
                gwait(cur, c0, rows_a, ga)
                scat(cur, c0, rows_a)
                gather(cur, c0 + 2, rows_a, ga)
                gwait(cur, c0 + 1, rows_b, gb)
                scat(cur, c0 + 1, rows_b)
                return c2

            lax.fori_loop(0, pairs, pair_body, 0)
            gwait(cur, cs - 1, rows_a, ga)
            scat(cur, cs - 1, rows_a)
            if s + 1 < sb:
                idx_wait(s + 1, 1 - cur)
                gather(1 - cur, 0, rows_a, ga)

        plsc.subcore_barrier()
        pltpu.sync_copy(accum_sh.at[stripe], out_hbm.at[cid, stripe])

    return agg_kernel


def _dis(degp_blk):
    deg = jnp.sum(degp_blk, axis=0) + 1.0
    return lax.rsqrt(deg)[:, None]


def _prep_body(x_ref, fi_ref, w1_ref, degp_ref, out_ref):
    xw = x_ref[...] * jax.nn.sigmoid(fi_ref[...])[0][None, :]
    h = jnp.dot(xw, w1_ref[...], preferred_element_type=jnp.float32)
    out_ref[...] = h * _dis(degp_ref[...])


def _mid_body(agg_ref, table_ref, degp_ref, b1_ref,
              bnw_ref, bnb_ref, w2_ref, out_ref):
    dis = _dis(degp_ref[...])
    agg = agg_ref[0] + agg_ref[1] - table_ref[...]
    t = dis * agg + b1_ref[...][0][None, :]
    t = t * bnw_ref[...][0][None, :] + bnb_ref[...][0][None, :]
    t = jnp.maximum(t, 0.0)
    g = jnp.dot(t, w2_ref[...], preferred_element_type=jnp.float32)
    out_ref[...] = g * dis


def _final_body(agg_ref, table_ref, degp_ref, b2_ref, out_ref):
    agg = agg_ref[0] + agg_ref[1] - table_ref[...]
    out_ref[...] = _dis(degp_ref[...]) * agg + b2_ref[...][0][None, :]


W2PAD = 16


def kernel(x, edge_index, feature_importance, W1, b1, gamma, beta,
           running_mean, running_var, W2, b2):
    n, d = x.shape
    e = edge_index.shape[1]
    h = W1.shape[1]
    out_dim = W2.shape[1]
    src = edge_index[0].astype(jnp.int32)
    dst = edge_index[1].astype(jnp.int32)

    row_align = NS * 8 * 8
    npad = ((n + row_align - 1) // row_align) * row_align

    k1, sb1 = 80, 5
    k2, sb2 = 2000, 1
    src1 = src.reshape(NW, sb1, e // (NW * k1 * sb1), k1)
    dst1 = dst.reshape(NW, sb1, e // (NW * k1 * sb1), k1)
    src2 = src.reshape(NW, sb2, e // (NW * k2 * sb2), k2)
    dst2 = dst.reshape(NW, sb2, e // (NW * k2 * sb2), k2)

    degp = _make_deg_kernel(npad, e)(dst).reshape(NW, npad)

    bn = 2048
    grid = npad // bn

    h1p = pl.pallas_call(
        _prep_body,
        grid=(grid,),
        in_specs=[
            pl.BlockSpec((bn, d), lambda i: (i, 0)),
            pl.BlockSpec((1, d), lambda i: (0, 0)),
            pl.BlockSpec((d, h), lambda i: (0, 0)),
            pl.BlockSpec((NW, bn), lambda i: (0, i)),
        ],
        out_specs=pl.BlockSpec((bn, h), lambda i: (i, 0)),
        out_shape=jax.ShapeDtypeStruct((npad, h), jnp.float32),
    )(x, feature_importance[None, :], W1, degp)

    agg1 = _make_agg_kernel(npad, e, h, k1, sb1)(h1p, src1, dst1)

    w2p = jnp.zeros((h, W2PAD), jnp.float32).at[:, :out_dim].set(W2)
    bnw = gamma * lax.rsqrt(running_var + 1e-5)
    bnb = beta - running_mean * bnw
    gp = pl.pallas_call(
        _mid_body,
        grid=(grid,),
        in_specs=[
            pl.BlockSpec((NC, bn, h), lambda i: (0, i, 0)),
            pl.BlockSpec((bn, h), lambda i: (i, 0)),
            pl.BlockSpec((NW, bn), lambda i: (0, i)),
            pl.BlockSpec((1, h), lambda i: (0, 0)),
            pl.BlockSpec((1, h), lambda i: (0, 0)),
            pl.BlockSpec((1, h), lambda i: (0, 0)),
            pl.BlockSpec((h, W2PAD), lambda i: (0, 0)),
        ],
        out_specs=pl.BlockSpec((bn, W2PAD), lambda i: (i, 0)),
        out_shape=jax.ShapeDtypeStruct((npad, W2PAD), jnp.float32),
    )(agg1, h1p, degp, b1[None, :], bnw[None, :], bnb[None, :], w2p)

    agg2 = _make_agg_kernel(npad, e, W2PAD, k2, sb2)(gp, src2, dst2)

    b2p = jnp.zeros((W2PAD,), jnp.float32).at[:out_dim].set(b2)
    outp = pl.pallas_call(
        _final_body,
        grid=(grid,),
        in_specs=[
            pl.BlockSpec((NC, bn, W2PAD), lambda i: (0, i, 0)),
            pl.BlockSpec((bn, W2PAD), lambda i: (i, 0)),
            pl.BlockSpec((NW, bn), lambda i: (0, i)),
            pl.BlockSpec((1, W2PAD), lambda i: (0, 0)),
        ],
        out_specs=pl.BlockSpec((bn, W2PAD), lambda i: (i, 0)),
        out_shape=jax.ShapeDtypeStruct((npad, W2PAD), jnp.float32),
    )(agg2, gp, degp, b2p[None, :])

    return outp[:n, :out_dim]
